# Initial kernel scaffold; baseline (speedup 1.0000x reference)
import jax
import jax.numpy as jnp
from jax.experimental import pallas as pl


def kernel(X, W_enc, b_enc, D, latent_bias, pre_bias):
    raise NotImplementedError("write your pallas kernel here")


# speedup vs baseline: 21.0427x; 21.0427x over previous
import functools

import jax
import jax.numpy as jnp
from jax.experimental import pallas as pl
from jax.experimental.pallas import tpu as pltpu

_TB = 256
_K = 64


def _inv_norm_kernel(d_ref, out_ref):
    d = d_ref[...]
    out_ref[...] = jax.lax.rsqrt(jnp.sum(d * d, axis=0, keepdims=True))


def _decode_key(k):
    neg = k < 0
    bits = jnp.where(neg, jnp.bitwise_xor(jnp.bitwise_not(k), jnp.int32(-(2**31))), k)
    return jax.lax.bitcast_convert_type(bits, jnp.float32)


def _main_kernel(x_ref, w_ref, bias_ref, pb_ref, d_ref, invn_ref, s_ref, xr_ref, *, k):
    xc = x_ref[...] - pb_ref[...]
    sp = jnp.dot(xc.astype(jnp.bfloat16), w_ref[...],
                 preferred_element_type=jnp.float32)
    sp = sp + bias_ref[...]

    tb = sp.shape[0]
    lo0 = jnp.full((tb, 1), jnp.int32(-(2**31)))
    hi0 = jnp.full((tb, 1), jnp.int32(2**31 - 1))

    def body(_, carry):
        lo, hi = carry
        mid = (lo >> 1) + (hi >> 1) + (lo & hi & 1)
        fmid = _decode_key(mid)
        cnt = jnp.sum((sp >= fmid).astype(jnp.int32), axis=1, keepdims=True)
        ge = cnt >= k
        return jnp.where(ge, mid, lo), jnp.where(ge, hi, mid)

    lo, _ = jax.lax.fori_loop(0, 32, body, (lo0, hi0))
    thresh = _decode_key(lo)

    s = jnp.where(sp >= thresh, jnp.maximum(sp, 0.0), 0.0)
    s_ref[...] = s
    xr = jnp.dot(s.astype(jnp.bfloat16), d_ref[...],
                 preferred_element_type=jnp.float32)
    xr_ref[...] = xr * invn_ref[...] + pb_ref[...]


def kernel(X, W_enc, b_enc, D, latent_bias, pre_bias):
    T, M = X.shape
    L = W_enc.shape[1]

    inv_norm = pl.pallas_call(
        _inv_norm_kernel,
        out_shape=jax.ShapeDtypeStruct((1, M), jnp.float32),
        in_specs=[pl.BlockSpec((L, M), lambda: (0, 0))],
        out_specs=pl.BlockSpec((1, M), lambda: (0, 0)),
    )(D)

    bias = (b_enc + latent_bias).reshape(1, L)
    pb = pre_bias.reshape(1, M)
    w16 = W_enc.astype(jnp.bfloat16)
    d16 = D.astype(jnp.bfloat16)

    grid = (T // _TB,)
    S, X_recon = pl.pallas_call(
        functools.partial(_main_kernel, k=_K),
        grid=grid,
        in_specs=[
            pl.BlockSpec((_TB, M), lambda i: (i, 0)),
            pl.BlockSpec((M, L), lambda i: (0, 0)),
            pl.BlockSpec((1, L), lambda i: (0, 0)),
            pl.BlockSpec((1, M), lambda i: (0, 0)),
            pl.BlockSpec((L, M), lambda i: (0, 0)),
            pl.BlockSpec((1, M), lambda i: (0, 0)),
        ],
        out_specs=[
            pl.BlockSpec((_TB, L), lambda i: (i, 0)),
            pl.BlockSpec((_TB, M), lambda i: (i, 0)),
        ],
        out_shape=[
            jax.ShapeDtypeStruct((T, L), jnp.float32),
            jax.ShapeDtypeStruct((T, M), jnp.float32),
        ],
        compiler_params=pltpu.CompilerParams(
            dimension_semantics=("parallel",)),
    )(X, w16, bias, pb, d16, inv_norm)
    return (S, X_recon)


# speedup vs baseline: 27.8411x; 1.3231x over previous
import functools

import jax
import jax.numpy as jnp
from jax.experimental import pallas as pl
from jax.experimental.pallas import tpu as pltpu

_TB = 256
_K = 64


def _inv_norm_kernel(d_ref, out_ref):
    d = d_ref[...]
    out_ref[...] = jax.lax.rsqrt(jnp.sum(d * d, axis=0, keepdims=True))


def _decode_key(k):
    neg = k < 0
    bits = jnp.where(neg, jnp.bitwise_xor(jnp.bitwise_not(k), jnp.int32(-(2**31))), k)
    return jax.lax.bitcast_convert_type(bits, jnp.float32)


def _encode_key(x):
    b = jax.lax.bitcast_convert_type(x, jnp.int32)
    return jnp.where(b >= 0, b,
                     jnp.bitwise_xor(jnp.bitwise_not(b), jnp.int32(-(2**31))))


def _main_kernel(x_ref, w_ref, bias_ref, pb_ref, d_ref, invn_ref, s_ref, xr_ref, *, k):
    xc = x_ref[...] - pb_ref[...]
    sp = jnp.dot(xc.astype(jnp.bfloat16), w_ref[...],
                 preferred_element_type=jnp.float32)
    sp = sp + bias_ref[...]

    tb = sp.shape[0]
    nl = sp.shape[1]
    cm = jnp.max(sp.reshape(tb, 64, nl // 64), axis=2)
    ub = jnp.max(cm, axis=1, keepdims=True)
    lb = jnp.min(cm, axis=1, keepdims=True)
    lo0 = _encode_key(lb)
    hi0 = _encode_key(ub) + 1

    def body(_, carry):
        lo, hi = carry
        mid = (lo >> 1) + (hi >> 1) + (lo & hi & 1)
        fmid = _decode_key(mid)
        cnt = jnp.sum((sp >= fmid).astype(jnp.int32), axis=1, keepdims=True)
        ge = cnt >= k
        return jnp.where(ge, mid, lo), jnp.where(ge, hi, mid)

    lo, _ = jax.lax.fori_loop(0, 20, body, (lo0, hi0))
    thresh = _decode_key(lo)

    s = jnp.where(sp >= thresh, jnp.maximum(sp, 0.0), 0.0)
    s_ref[...] = s
    xr = jnp.dot(s.astype(jnp.bfloat16), d_ref[...],
                 preferred_element_type=jnp.float32)
    xr_ref[...] = xr * invn_ref[...] + pb_ref[...]


def kernel(X, W_enc, b_enc, D, latent_bias, pre_bias):
    T, M = X.shape
    L = W_enc.shape[1]

    inv_norm = pl.pallas_call(
        _inv_norm_kernel,
        out_shape=jax.ShapeDtypeStruct((1, M), jnp.float32),
        in_specs=[pl.BlockSpec((L, M), lambda: (0, 0))],
        out_specs=pl.BlockSpec((1, M), lambda: (0, 0)),
    )(D)

    bias = (b_enc + latent_bias).reshape(1, L)
    pb = pre_bias.reshape(1, M)
    w16 = W_enc.astype(jnp.bfloat16)
    d16 = D.astype(jnp.bfloat16)

    grid = (T // _TB,)
    S, X_recon = pl.pallas_call(
        functools.partial(_main_kernel, k=_K),
        grid=grid,
        in_specs=[
            pl.BlockSpec((_TB, M), lambda i: (i, 0)),
            pl.BlockSpec((M, L), lambda i: (0, 0)),
            pl.BlockSpec((1, L), lambda i: (0, 0)),
            pl.BlockSpec((1, M), lambda i: (0, 0)),
            pl.BlockSpec((L, M), lambda i: (0, 0)),
            pl.BlockSpec((1, M), lambda i: (0, 0)),
        ],
        out_specs=[
            pl.BlockSpec((_TB, L), lambda i: (i, 0)),
            pl.BlockSpec((_TB, M), lambda i: (i, 0)),
        ],
        out_shape=[
            jax.ShapeDtypeStruct((T, L), jnp.float32),
            jax.ShapeDtypeStruct((T, M), jnp.float32),
        ],
        compiler_params=pltpu.CompilerParams(
            dimension_semantics=("parallel",)),
    )(X, w16, bias, pb, d16, inv_norm)
    return (S, X_recon)
